# baseline (speedup 1.0000x reference)
import functools

import jax
import jax.numpy as jnp
from jax import lax
from jax.experimental import pallas as pl
from jax.experimental.pallas import tpu as pltpu
from jax.experimental.pallas import tpu_sc as plsc

N = 10000
E = 320000
D_FEAT = 128
D_EDGE = 16
D_OUT = 16

NUM_CORES = 2
NUM_SUBCORES = 16
NUM_WORKERS = NUM_CORES * NUM_SUBCORES
CHUNK = 1280
SLABS = CHUNK // 128
TOTAL_CHUNKS = E // CHUNK
MAX_CHUNKS_PER_W = -(-TOTAL_CHUNKS // NUM_WORKERS)
FINAL_BLK = 12800


def _proj_body(node_ref, w_ref, b_ref, ps_ref, pd_ref):
    p = jnp.dot(node_ref[...], w_ref[...], preferred_element_type=jnp.float32)
    ps_ref[...] = p[:, :D_OUT] + b_ref[...]
    pd_ref[...] = p[:, D_OUT:]


def _final_body(eat_ref, g_ref, w_ref, o_ref):
    gt = jnp.concatenate(
        [g_ref[t] for t in range(FINAL_BLK // 128)], axis=1)
    o_ref[...] = (
        jnp.dot(w_ref[...], eat_ref[...], preferred_element_type=jnp.float32)
        + gt
    )


def _sc_body(ps_hbm, pd_hbm, ei_hbm, gsum_hbm,
             idx_s0, idx_s1, idx_d0, idx_d1,
             rows_s0, rows_s1, rows_d0, rows_d1, res_v,
             ps_sh, pd_sh,
             sem_i0, sem_i1, sem_g0, sem_g1, sem_o):
    idx_s = [idx_s0, idx_s1]
    idx_d = [idx_d0, idx_d1]
    rows_s = [rows_s0, rows_s1]
    rows_d = [rows_d0, rows_d1]
    sem_i = [sem_i0, sem_i1]
    sem_g = [sem_g0, sem_g1]
    wid = lax.axis_index("c") * NUM_SUBCORES + lax.axis_index("s")
    lanes = lax.iota(jnp.int32, 16)

    def ci_of(j):
        return wid + NUM_WORKERS * j

    def valid(j):
        return ci_of(j) < TOTAL_CHUNKS

    def issue_idx(j):
        b = j % 2

        @pl.when(valid(j))
        def _():
            off = ci_of(j) * CHUNK
            pltpu.async_copy(ei_hbm.at[0, pl.ds(off, CHUNK)], idx_s[b], sem_i[b])
            pltpu.async_copy(ei_hbm.at[1, pl.ds(off, CHUNK)], idx_d[b], sem_i[b])

    def issue_gather(j):
        b = j % 2

        @pl.when(valid(j))
        def _():
            off = ci_of(j) * CHUNK
            pltpu.make_async_copy(
                ei_hbm.at[0, pl.ds(off, CHUNK)], idx_s[b], sem_i[b]).wait()
            pltpu.make_async_copy(
                ei_hbm.at[1, pl.ds(off, CHUNK)], idx_d[b], sem_i[b]).wait()
            pltpu.async_copy(ps_sh.at[idx_s[b]], rows_s[b], sem_g[b])
            pltpu.async_copy(pd_sh.at[idx_d[b]], rows_d[b], sem_g[b])

    def wait_out(j):
        for t in range(SLABS):
            pltpu.make_async_copy(
                res_v.at[t, :, pl.ds(0, 128)],
                gsum_hbm.at[ci_of(j) * SLABS + t], sem_o).wait()

    def process(j):
        b = j % 2

        @pl.when(valid(j))
        def _():
            pltpu.make_async_copy(ps_sh.at[idx_s[b]], rows_s[b], sem_g[b]).wait()
            pltpu.make_async_copy(pd_sh.at[idx_d[b]], rows_d[b], sem_g[b]).wait()
            if j > 0:
                wait_out(j - 1)

            @pl.loop(0, SLABS)
            def _(t):
                t_ids = jnp.full((16,), t, jnp.int32)

                @plsc.parallel_loop(0, 128, unroll=16)
                def _(i):
                    e = t * 128 + i
                    v = rows_s[b][e] + rows_d[b][e]
                    col = jnp.full((16,), i, jnp.int32)
                    plsc.store_scatter(res_v, [t_ids, lanes, col], v)

            for t in range(SLABS):
                pltpu.async_copy(
                    res_v.at[t, :, pl.ds(0, 128)],
                    gsum_hbm.at[ci_of(j) * SLABS + t], sem_o)

    issue_idx(0)
    issue_idx(1)

    @pl.when(lax.axis_index("s") == 0)
    def _():
        pltpu.sync_copy(ps_hbm, ps_sh)
        pltpu.sync_copy(pd_hbm, pd_sh)

    plsc.subcore_barrier()
    issue_gather(0)
    for j in range(MAX_CHUNKS_PER_W):
        if j + 1 < MAX_CHUNKS_PER_W:
            issue_gather(j + 1)
        process(j)
        if j + 2 < MAX_CHUNKS_PER_W:
            issue_idx(j + 2)

    last = MAX_CHUNKS_PER_W - 1

    @pl.when(valid(last))
    def _():
        wait_out(last)

    @pl.when(jnp.logical_not(valid(last)))
    def _():
        wait_out(last - 1)


def kernel(node_attr, edge_attr, edge_index, W, b):
    ei32 = edge_index.astype(jnp.int32)
    W = W.astype(jnp.float32)
    w_sd = jnp.concatenate([W[:D_FEAT], W[D_FEAT:2 * D_FEAT]], axis=1)
    w_et = W[2 * D_FEAT:].T
    b2 = b.astype(jnp.float32).reshape(1, D_OUT)

    ps, pd = pl.pallas_call(
        _proj_body,
        out_shape=(
            jax.ShapeDtypeStruct((N, D_OUT), jnp.float32),
            jax.ShapeDtypeStruct((N, D_OUT), jnp.float32),
        ),
    )(node_attr, w_sd, b2)

    mesh = plsc.VectorSubcoreMesh(
        core_axis_name="c", subcore_axis_name="s",
        num_cores=NUM_CORES, num_subcores=NUM_SUBCORES,
    )
    sc = functools.partial(
        pl.kernel,
        mesh=mesh,
        compiler_params=pltpu.CompilerParams(
            use_tc_tiling_on_sc=False, needs_layout_passes=False),
        out_type=jax.ShapeDtypeStruct((E // 128, D_OUT, 128), jnp.float32),
        scratch_types=[
            pltpu.VMEM((CHUNK,), jnp.int32),
            pltpu.VMEM((CHUNK,), jnp.int32),
            pltpu.VMEM((CHUNK,), jnp.int32),
            pltpu.VMEM((CHUNK,), jnp.int32),
            pltpu.VMEM((CHUNK, D_OUT), jnp.float32),
            pltpu.VMEM((CHUNK, D_OUT), jnp.float32),
            pltpu.VMEM((CHUNK, D_OUT), jnp.float32),
            pltpu.VMEM((CHUNK, D_OUT), jnp.float32),
            pltpu.VMEM((SLABS, D_OUT, 129), jnp.float32),
            pltpu.VMEM_SHARED((N, D_OUT), jnp.float32),
            pltpu.VMEM_SHARED((N, D_OUT), jnp.float32),
            pltpu.SemaphoreType.DMA,
            pltpu.SemaphoreType.DMA,
            pltpu.SemaphoreType.DMA,
            pltpu.SemaphoreType.DMA,
            pltpu.SemaphoreType.DMA,
        ],
    )(_sc_body)
    gsum = sc(ps, pd, ei32)

    out_t = pl.pallas_call(
        _final_body,
        grid=(E // FINAL_BLK,),
        in_specs=[
            pl.BlockSpec((D_EDGE, FINAL_BLK), lambda i: (0, i)),
            pl.BlockSpec((FINAL_BLK // 128, D_OUT, 128), lambda i: (i, 0, 0)),
            pl.BlockSpec((D_OUT, D_EDGE), lambda i: (0, 0)),
        ],
        out_specs=pl.BlockSpec((D_OUT, FINAL_BLK), lambda i: (0, i)),
        out_shape=jax.ShapeDtypeStruct((D_OUT, E), jnp.float32),
    )(edge_attr.T, gsum, w_et)
    return out_t.T


# speedup vs baseline: 1.0026x; 1.0026x over previous
import functools

import jax
import jax.numpy as jnp
from jax import lax
from jax.experimental import pallas as pl
from jax.experimental.pallas import tpu as pltpu
from jax.experimental.pallas import tpu_sc as plsc

N = 10000
E = 320000
D_FEAT = 128
D_EDGE = 16
D_OUT = 16

NUM_CORES = 2
NUM_SUBCORES = 16
NUM_WORKERS = NUM_CORES * NUM_SUBCORES
CHUNK = 1280
SLABS = CHUNK // 128
TOTAL_CHUNKS = E // CHUNK
MAX_CHUNKS_PER_W = -(-TOTAL_CHUNKS // NUM_WORKERS)
FINAL_BLK = 12800


def _proj_body(node_ref, w_ref, b_ref, ps_ref, pd_ref):
    p = jnp.dot(node_ref[...], w_ref[...], preferred_element_type=jnp.float32)
    ps_ref[...] = p[:, :D_OUT] + b_ref[...]
    pd_ref[...] = p[:, D_OUT:]


def _final_body(eat_ref, g_ref, w_ref, o_ref):
    gt = jnp.concatenate(
        [g_ref[t] for t in range(FINAL_BLK // 128)], axis=1)
    o_ref[...] = (
        jnp.dot(w_ref[...], eat_ref[...], preferred_element_type=jnp.float32)
        + gt
    )


def _sc_body(ps_hbm, pd_hbm, ei_hbm, gsum_hbm,
             idx_s0, idx_s1, idx_d0, idx_d1,
             rows_s0, rows_s1, rows_d0, rows_d1, res_v,
             ps_sh, pd_sh,
             sem_i0, sem_i1, sem_g0, sem_g1, sem_o):
    idx_s = [idx_s0, idx_s1]
    idx_d = [idx_d0, idx_d1]
    rows_s = [rows_s0, rows_s1]
    rows_d = [rows_d0, rows_d1]
    sem_i = [sem_i0, sem_i1]
    sem_g = [sem_g0, sem_g1]
    wid = lax.axis_index("c") * NUM_SUBCORES + lax.axis_index("s")
    lanes = lax.iota(jnp.int32, 16)

    def ci_of(j):
        return wid + NUM_WORKERS * j

    def valid(j):
        return ci_of(j) < TOTAL_CHUNKS

    def issue_idx(j):
        b = j % 2

        @pl.when(valid(j))
        def _():
            off = ci_of(j) * CHUNK
            pltpu.async_copy(ei_hbm.at[0, pl.ds(off, CHUNK)], idx_s[b], sem_i[b])
            pltpu.async_copy(ei_hbm.at[1, pl.ds(off, CHUNK)], idx_d[b], sem_i[b])

    def issue_gather(j):
        b = j % 2

        @pl.when(valid(j))
        def _():
            off = ci_of(j) * CHUNK
            pltpu.make_async_copy(
                ei_hbm.at[0, pl.ds(off, CHUNK)], idx_s[b], sem_i[b]).wait()
            pltpu.make_async_copy(
                ei_hbm.at[1, pl.ds(off, CHUNK)], idx_d[b], sem_i[b]).wait()
            pltpu.async_copy(ps_sh.at[idx_s[b]], rows_s[b], sem_g[b])
            pltpu.async_copy(pd_sh.at[idx_d[b]], rows_d[b], sem_g[b])

    def wait_out(j):
        for t in range(SLABS):
            pltpu.make_async_copy(
                res_v.at[t, :, pl.ds(0, 128)],
                gsum_hbm.at[ci_of(j) * SLABS + t], sem_o).wait()

    def process(j):
        b = j % 2

        @pl.when(valid(j))
        def _():
            pltpu.make_async_copy(ps_sh.at[idx_s[b]], rows_s[b], sem_g[b]).wait()
            pltpu.make_async_copy(pd_sh.at[idx_d[b]], rows_d[b], sem_g[b]).wait()
            if j > 0:
                wait_out(j - 1)

            @pl.loop(0, SLABS)
            def _(t):
                t_ids = jnp.full((16,), t, jnp.int32)

                @plsc.parallel_loop(0, 128, unroll=8)
                def _(i):
                    e = t * 128 + i
                    v = rows_s[b][e] + rows_d[b][e]
                    col = jnp.full((16,), i, jnp.int32)
                    plsc.store_scatter(res_v, [t_ids, lanes, col], v)

            for t in range(SLABS):
                pltpu.async_copy(
                    res_v.at[t, :, pl.ds(0, 128)],
                    gsum_hbm.at[ci_of(j) * SLABS + t], sem_o)

    issue_idx(0)
    issue_idx(1)

    @pl.when(lax.axis_index("s") == 0)
    def _():
        pltpu.sync_copy(ps_hbm, ps_sh)
        pltpu.sync_copy(pd_hbm, pd_sh)

    plsc.subcore_barrier()
    issue_gather(0)
    for j in range(MAX_CHUNKS_PER_W):
        if j + 1 < MAX_CHUNKS_PER_W:
            issue_gather(j + 1)
        process(j)
        if j + 2 < MAX_CHUNKS_PER_W:
            issue_idx(j + 2)

    last = MAX_CHUNKS_PER_W - 1

    @pl.when(valid(last))
    def _():
        wait_out(last)

    @pl.when(jnp.logical_not(valid(last)))
    def _():
        wait_out(last - 1)


def kernel(node_attr, edge_attr, edge_index, W, b):
    ei32 = edge_index.astype(jnp.int32)
    W = W.astype(jnp.float32)
    w_sd = jnp.concatenate([W[:D_FEAT], W[D_FEAT:2 * D_FEAT]], axis=1)
    w_et = W[2 * D_FEAT:].T
    b2 = b.astype(jnp.float32).reshape(1, D_OUT)

    ps, pd = pl.pallas_call(
        _proj_body,
        out_shape=(
            jax.ShapeDtypeStruct((N, D_OUT), jnp.float32),
            jax.ShapeDtypeStruct((N, D_OUT), jnp.float32),
        ),
    )(node_attr, w_sd, b2)

    mesh = plsc.VectorSubcoreMesh(
        core_axis_name="c", subcore_axis_name="s",
        num_cores=NUM_CORES, num_subcores=NUM_SUBCORES,
    )
    sc = functools.partial(
        pl.kernel,
        mesh=mesh,
        compiler_params=pltpu.CompilerParams(
            use_tc_tiling_on_sc=False, needs_layout_passes=False),
        out_type=jax.ShapeDtypeStruct((E // 128, D_OUT, 128), jnp.float32),
        scratch_types=[
            pltpu.VMEM((CHUNK,), jnp.int32),
            pltpu.VMEM((CHUNK,), jnp.int32),
            pltpu.VMEM((CHUNK,), jnp.int32),
            pltpu.VMEM((CHUNK,), jnp.int32),
            pltpu.VMEM((CHUNK, D_OUT), jnp.float32),
            pltpu.VMEM((CHUNK, D_OUT), jnp.float32),
            pltpu.VMEM((CHUNK, D_OUT), jnp.float32),
            pltpu.VMEM((CHUNK, D_OUT), jnp.float32),
            pltpu.VMEM((SLABS, D_OUT, 129), jnp.float32),
            pltpu.VMEM_SHARED((N, D_OUT), jnp.float32),
            pltpu.VMEM_SHARED((N, D_OUT), jnp.float32),
            pltpu.SemaphoreType.DMA,
            pltpu.SemaphoreType.DMA,
            pltpu.SemaphoreType.DMA,
            pltpu.SemaphoreType.DMA,
            pltpu.SemaphoreType.DMA,
        ],
    )(_sc_body)
    gsum = sc(ps, pd, ei32)

    out_t = pl.pallas_call(
        _final_body,
        grid=(E // FINAL_BLK,),
        in_specs=[
            pl.BlockSpec((D_EDGE, FINAL_BLK), lambda i: (0, i)),
            pl.BlockSpec((FINAL_BLK // 128, D_OUT, 128), lambda i: (i, 0, 0)),
            pl.BlockSpec((D_OUT, D_EDGE), lambda i: (0, 0)),
        ],
        out_specs=pl.BlockSpec((D_OUT, FINAL_BLK), lambda i: (0, i)),
        out_shape=jax.ShapeDtypeStruct((D_OUT, E), jnp.float32),
    )(edge_attr.T, gsum, w_et)
    return out_t.T
